# Initial kernel scaffold; baseline (speedup 1.0000x reference)
import functools

import jax
import jax.numpy as jnp
from jax.experimental import pallas as pl

N = 50000
K = 256
Q_MIN = 0.1
S_B = 1.0
E_DEN_OFF = 1.0
BLK = 1000
NB = N // BLK
BIG = jnp.int32(2 ** 30)


def _a1_body(tidx_ref, beta_ref, segmax_ref, pwsum_ref, nstat_ref):
    pid = pl.program_id(0)

    @pl.when(pid == 0)
    def _():
        segmax_ref[...] = jnp.full((1, K), -1.0, jnp.float32)
        pwsum_ref[...] = jnp.zeros((1, K), jnp.float32)
        nstat_ref[...] = jnp.zeros((1, 128), jnp.float32)

    tidx = tidx_ref[...]
    beta_raw = beta_ref[...]
    beta = jnp.clip(beta_raw, 1e-6, 1.0 - 1e-4)
    is_noise = tidx < 0
    obj = jnp.where(is_noise, 0, tidx)
    beta_m = jnp.where(is_noise, -1.0, beta)
    cols = jax.lax.broadcasted_iota(jnp.int32, (1, K), 1)
    m = obj == cols
    segmax_ref[...] = jnp.maximum(
        segmax_ref[...],
        jnp.max(jnp.where(m, beta_m, -1.0), axis=0, keepdims=True))
    pw = jnp.where(is_noise, 0.0, beta * beta)
    pwsum_ref[...] += jnp.sum(jnp.where(m, pw, 0.0), axis=0, keepdims=True)
    nf = is_noise.astype(jnp.float32)
    stat = nstat_ref[...]
    stat = stat.at[0, 0].add(jnp.sum(nf))
    stat = stat.at[0, 1].add(jnp.sum(nf * beta))
    nstat_ref[...] = stat


def _a2_body(tidx_ref, beta_ref, segmax_ref, aidx_ref):
    pid = pl.program_id(0)

    @pl.when(pid == 0)
    def _():
        aidx_ref[...] = jnp.full((1, K), BIG, jnp.int32)

    tidx = tidx_ref[...]
    beta_raw = beta_ref[...]
    beta = jnp.clip(beta_raw, 1e-6, 1.0 - 1e-4)
    is_noise = tidx < 0
    obj = jnp.where(is_noise, 0, tidx)
    beta_m = jnp.where(is_noise, -1.0, beta)
    cols = jax.lax.broadcasted_iota(jnp.int32, (1, K), 1)
    m = obj == cols
    smax = jnp.sum(jnp.where(m, segmax_ref[...], 0.0), axis=1, keepdims=True)
    is_alpha = jnp.logical_and(beta_m >= smax, jnp.logical_not(is_noise))
    rows = jax.lax.broadcasted_iota(jnp.int32, (BLK, 1), 0)
    gidx = rows + pid * BLK
    key = jnp.where(is_alpha, gidx, BIG)
    aidx_ref[...] = jnp.minimum(
        aidx_ref[...],
        jnp.min(jnp.where(m, key, BIG), axis=0, keepdims=True))


def _d_body(tidx_ref, beta_ref, cc_ref, en_ref, pos_ref, tim_ref,
            ten_ref, tpos_ref, ttim_ref,
            xa0_ref, xa1_ref, qa_ref, pwsum_ref,
            acc_ref):
    pid = pl.program_id(0)

    @pl.when(pid == 0)
    def _():
        acc_ref[...] = jnp.zeros((1, 128), jnp.float32)

    tidx = tidx_ref[...]
    beta_raw = beta_ref[...]
    beta = jnp.clip(beta_raw, 1e-6, 1.0 - 1e-4)
    q = 0.25 * jnp.log((1.0 + beta) / (1.0 - beta)) ** 2 + Q_MIN
    is_noise = tidx < 0
    obj = jnp.where(is_noise, 0, tidx)
    cols = jax.lax.broadcasted_iota(jnp.int32, (1, K), 1)
    m = jnp.logical_and(obj == cols, jnp.logical_not(is_noise))

    cc = cc_ref[...]
    x0 = cc[:, 0:1]
    x1 = cc[:, 1:2]
    dx0 = x0 - xa0_ref[...]
    dx1 = x1 - xa1_ref[...]
    d2 = dx0 * dx0 + dx1 * dx1
    dist = jnp.sqrt(d2 + 1e-9)
    qq = q * qa_ref[...]
    v_att = jnp.sum(jnp.where(m, qq * d2, 0.0))
    v_rep = jnp.sum(jnp.where(m, 0.0, qq * jnp.maximum(1.0 - dist, 0.0)))

    pw = jnp.where(is_noise, 0.0, beta * beta)
    pws = jnp.sum(jnp.where(m, pwsum_ref[...], 0.0), axis=1, keepdims=True)
    pnorm = pw / (pws + 1e-9)
    en = en_ref[...]
    ten = ten_ref[...]
    e_l = ((en - ten) / (jnp.abs(ten) + E_DEN_OFF)) ** 2
    dp = pos_ref[...] - tpos_ref[...]
    pos_d = jnp.sqrt(jnp.sum(dp * dp, axis=1, keepdims=True) + 1e-6)
    pos_l = jnp.where(pos_d < 10.0, pos_d * pos_d,
                      100.0 + 20.0 * (pos_d - 10.0))
    tim_l = (tim_ref[...] - ttim_ref[...]) ** 2

    acc = acc_ref[...]
    acc = acc.at[0, 0].add(v_att)
    acc = acc.at[0, 1].add(v_rep)
    acc = acc.at[0, 2].add(jnp.sum(pnorm * e_l))
    acc = acc.at[0, 3].add(jnp.sum(pnorm * pos_l))
    acc = acc.at[0, 4].add(jnp.sum(pnorm * tim_l))
    acc_ref[...] = acc


def _row_spec(width):
    return pl.BlockSpec((BLK, width), lambda i: (i, 0))


_FULL_K = pl.BlockSpec((1, K), lambda i: (0, 0))
_FULL_S = pl.BlockSpec((1, 128), lambda i: (0, 0))


@jax.jit
def _run(pred_beta, pred_ccoords, pred_energy, pred_pos, pred_time,
         t_idx, t_energy, t_pos, t_time):
    segmax, pwsum, nstat = pl.pallas_call(
        _a1_body,
        grid=(NB,),
        in_specs=[_row_spec(1), _row_spec(1)],
        out_specs=[_FULL_K, _FULL_K, _FULL_S],
        out_shape=[
            jax.ShapeDtypeStruct((1, K), jnp.float32),
            jax.ShapeDtypeStruct((1, K), jnp.float32),
            jax.ShapeDtypeStruct((1, 128), jnp.float32),
        ],
    )(t_idx, pred_beta)

    aidx = pl.pallas_call(
        _a2_body,
        grid=(NB,),
        in_specs=[_row_spec(1), _row_spec(1), _FULL_K],
        out_specs=_FULL_K,
        out_shape=jax.ShapeDtypeStruct((1, K), jnp.int32),
    )(t_idx, pred_beta, segmax)

    aidx_v = aidx[0]
    valid = (aidx_v < N).astype(jnp.float32)
    a = jnp.minimum(aidx_v, N - 1)
    x_a = pred_ccoords[a]
    beta_a = jnp.clip(pred_beta[a, 0], 1e-6, 1.0 - 1e-4)
    q_a = (jnp.arctanh(beta_a) ** 2 + Q_MIN) * valid

    acc = pl.pallas_call(
        _d_body,
        grid=(NB,),
        in_specs=[_row_spec(1), _row_spec(1), _row_spec(2), _row_spec(1),
                  _row_spec(2), _row_spec(1), _row_spec(1), _row_spec(2),
                  _row_spec(1), _FULL_K, _FULL_K, _FULL_K, _FULL_K],
        out_specs=_FULL_S,
        out_shape=jax.ShapeDtypeStruct((1, 128), jnp.float32),
    )(t_idx, pred_beta, pred_ccoords, pred_energy, pred_pos, pred_time,
      t_energy, t_pos, t_time,
      x_a[:, 0][None, :], x_a[:, 1][None, :], q_a[None, :], pwsum)

    nf = jnp.float32(N)
    v_att = acc[0, 0] / nf
    v_rep = acc[0, 1] / nf
    n_obj = jnp.maximum(jnp.sum(valid), 1.0)
    l_beta = jnp.sum((1.0 - beta_a) * valid) / n_obj
    n_noise = jnp.maximum(nstat[0, 0], 1.0)
    l_noise = S_B * nstat[0, 1] / n_noise
    l_e = acc[0, 2] / n_obj
    l_pos = acc[0, 3] / n_obj
    l_t = acc[0, 4] / n_obj
    total = v_att + v_rep + l_beta + l_noise + l_e + l_pos + l_t
    return jnp.reshape(total, [1])


def kernel(pred_beta, pred_ccoords, pred_energy, pred_pos, pred_time,
           rechit_energy, t_idx, t_energy, t_pos, t_time, row_splits):
    lossval = _run(pred_beta, pred_ccoords, pred_energy, pred_pos, pred_time,
                   t_idx, t_energy, t_pos, t_time)
    return (pred_beta, lossval)


# speedup vs baseline: 2.5378x; 2.5378x over previous
import functools

import jax
import jax.numpy as jnp
from jax.experimental import pallas as pl

N = 50000
K = 256
Q_MIN = 0.1
S_B = 1.0
E_DEN_OFF = 1.0
BLK = 1000
NB = N // BLK
BIG = 2 ** 30


def _a1_body(tidx_ref, beta_ref, segmax_ref, pwsum_ref, nstat_ref):
    pid = pl.program_id(0)

    @pl.when(pid == 0)
    def _():
        segmax_ref[...] = jnp.full((1, K), -1.0, jnp.float32)
        pwsum_ref[...] = jnp.zeros((1, K), jnp.float32)
        nstat_ref[...] = jnp.zeros((1, 128), jnp.float32)

    tidx = tidx_ref[...]
    beta_raw = beta_ref[...]
    beta = jnp.clip(beta_raw, 1e-6, 1.0 - 1e-4)
    is_noise = tidx < 0
    obj = jnp.where(is_noise, 0, tidx)
    beta_m = jnp.where(is_noise, -1.0, beta)
    cols = jax.lax.broadcasted_iota(jnp.int32, (1, K), 1)
    m = obj == cols
    segmax_ref[...] = jnp.maximum(
        segmax_ref[...],
        jnp.max(jnp.where(m, beta_m, -1.0), axis=0, keepdims=True))
    pw = jnp.where(is_noise, 0.0, beta * beta)
    pwsum_ref[...] += jnp.sum(jnp.where(m, pw, 0.0), axis=0, keepdims=True)
    nf = is_noise.astype(jnp.float32)
    lane = jax.lax.broadcasted_iota(jnp.int32, (1, 128), 1)
    upd = (jnp.where(lane == 0, jnp.sum(nf), 0.0)
           + jnp.where(lane == 1, jnp.sum(nf * beta), 0.0))
    nstat_ref[...] += upd


def _a2_body(tidx_ref, beta_ref, segmax_ref, aidx_ref):
    pid = pl.program_id(0)

    @pl.when(pid == 0)
    def _():
        aidx_ref[...] = jnp.full((1, K), BIG, jnp.int32)

    tidx = tidx_ref[...]
    beta_raw = beta_ref[...]
    beta = jnp.clip(beta_raw, 1e-6, 1.0 - 1e-4)
    is_noise = tidx < 0
    obj = jnp.where(is_noise, 0, tidx)
    beta_m = jnp.where(is_noise, -1.0, beta)
    cols = jax.lax.broadcasted_iota(jnp.int32, (1, K), 1)
    m = obj == cols
    smax = jnp.sum(jnp.where(m, segmax_ref[...], 0.0), axis=1, keepdims=True)
    is_alpha = jnp.logical_and(beta_m >= smax, jnp.logical_not(is_noise))
    rows = jax.lax.broadcasted_iota(jnp.int32, (BLK, 1), 0)
    gidx = rows + pid * BLK
    key = jnp.where(is_alpha, gidx, BIG)
    aidx_ref[...] = jnp.minimum(
        aidx_ref[...],
        jnp.min(jnp.where(m, key, BIG), axis=0, keepdims=True))


def _d_body(tidx_ref, beta_ref, cc_ref, en_ref, pos_ref, tim_ref,
            ten_ref, tpos_ref, ttim_ref,
            xa0_ref, xa1_ref, qa_ref, pwsum_ref,
            acc_ref):
    pid = pl.program_id(0)

    @pl.when(pid == 0)
    def _():
        acc_ref[...] = jnp.zeros((1, 128), jnp.float32)

    tidx = tidx_ref[...]
    beta_raw = beta_ref[...]
    beta = jnp.clip(beta_raw, 1e-6, 1.0 - 1e-4)
    q = 0.25 * jnp.log((1.0 + beta) / (1.0 - beta)) ** 2 + Q_MIN
    is_noise = tidx < 0
    obj = jnp.where(is_noise, 0, tidx)
    cols = jax.lax.broadcasted_iota(jnp.int32, (1, K), 1)
    m = jnp.logical_and(obj == cols, jnp.logical_not(is_noise))

    cc = cc_ref[...]
    x0 = cc[:, 0:1]
    x1 = cc[:, 1:2]
    dx0 = x0 - xa0_ref[...]
    dx1 = x1 - xa1_ref[...]
    d2 = dx0 * dx0 + dx1 * dx1
    dist = jnp.sqrt(d2 + 1e-9)
    qq = q * qa_ref[...]
    v_att = jnp.sum(jnp.where(m, qq * d2, 0.0))
    v_rep = jnp.sum(jnp.where(m, 0.0, qq * jnp.maximum(1.0 - dist, 0.0)))

    pw = jnp.where(is_noise, 0.0, beta * beta)
    pws = jnp.sum(jnp.where(m, pwsum_ref[...], 0.0), axis=1, keepdims=True)
    pnorm = pw / (pws + 1e-9)
    en = en_ref[...]
    ten = ten_ref[...]
    e_l = ((en - ten) / (jnp.abs(ten) + E_DEN_OFF)) ** 2
    dp = pos_ref[...] - tpos_ref[...]
    pos_d = jnp.sqrt(jnp.sum(dp * dp, axis=1, keepdims=True) + 1e-6)
    pos_l = jnp.where(pos_d < 10.0, pos_d * pos_d,
                      100.0 + 20.0 * (pos_d - 10.0))
    tim_l = (tim_ref[...] - ttim_ref[...]) ** 2

    lane = jax.lax.broadcasted_iota(jnp.int32, (1, 128), 1)
    upd = (jnp.where(lane == 0, v_att, 0.0)
           + jnp.where(lane == 1, v_rep, 0.0)
           + jnp.where(lane == 2, jnp.sum(pnorm * e_l), 0.0)
           + jnp.where(lane == 3, jnp.sum(pnorm * pos_l), 0.0)
           + jnp.where(lane == 4, jnp.sum(pnorm * tim_l), 0.0))
    acc_ref[...] += upd


def _row_spec(width):
    return pl.BlockSpec((BLK, width), lambda i: (i, 0))


_FULL_K = pl.BlockSpec((1, K), lambda i: (0, 0))
_FULL_S = pl.BlockSpec((1, 128), lambda i: (0, 0))


@jax.jit
def _run(pred_beta, pred_ccoords, pred_energy, pred_pos, pred_time,
         t_idx, t_energy, t_pos, t_time):
    segmax, pwsum, nstat = pl.pallas_call(
        _a1_body,
        grid=(NB,),
        in_specs=[_row_spec(1), _row_spec(1)],
        out_specs=[_FULL_K, _FULL_K, _FULL_S],
        out_shape=[
            jax.ShapeDtypeStruct((1, K), jnp.float32),
            jax.ShapeDtypeStruct((1, K), jnp.float32),
            jax.ShapeDtypeStruct((1, 128), jnp.float32),
        ],
    )(t_idx, pred_beta)

    aidx = pl.pallas_call(
        _a2_body,
        grid=(NB,),
        in_specs=[_row_spec(1), _row_spec(1), _FULL_K],
        out_specs=_FULL_K,
        out_shape=jax.ShapeDtypeStruct((1, K), jnp.int32),
    )(t_idx, pred_beta, segmax)

    aidx_v = aidx[0]
    valid = (aidx_v < N).astype(jnp.float32)
    a = jnp.minimum(aidx_v, N - 1)
    x_a = pred_ccoords[a]
    beta_a = jnp.clip(pred_beta[a, 0], 1e-6, 1.0 - 1e-4)
    q_a = (jnp.arctanh(beta_a) ** 2 + Q_MIN) * valid

    acc = pl.pallas_call(
        _d_body,
        grid=(NB,),
        in_specs=[_row_spec(1), _row_spec(1), _row_spec(2), _row_spec(1),
                  _row_spec(2), _row_spec(1), _row_spec(1), _row_spec(2),
                  _row_spec(1), _FULL_K, _FULL_K, _FULL_K, _FULL_K],
        out_specs=_FULL_S,
        out_shape=jax.ShapeDtypeStruct((1, 128), jnp.float32),
    )(t_idx, pred_beta, pred_ccoords, pred_energy, pred_pos, pred_time,
      t_energy, t_pos, t_time,
      x_a[:, 0][None, :], x_a[:, 1][None, :], q_a[None, :], pwsum)

    nf = jnp.float32(N)
    v_att = acc[0, 0] / nf
    v_rep = acc[0, 1] / nf
    n_obj = jnp.maximum(jnp.sum(valid), 1.0)
    l_beta = jnp.sum((1.0 - beta_a) * valid) / n_obj
    n_noise = jnp.maximum(nstat[0, 0], 1.0)
    l_noise = S_B * nstat[0, 1] / n_noise
    l_e = acc[0, 2] / n_obj
    l_pos = acc[0, 3] / n_obj
    l_t = acc[0, 4] / n_obj
    total = v_att + v_rep + l_beta + l_noise + l_e + l_pos + l_t
    return jnp.reshape(total, [1])


def kernel(pred_beta, pred_ccoords, pred_energy, pred_pos, pred_time,
           rechit_energy, t_idx, t_energy, t_pos, t_time, row_splits):
    lossval = _run(pred_beta, pred_ccoords, pred_energy, pred_pos, pred_time,
                   t_idx, t_energy, t_pos, t_time)
    return (pred_beta, lossval)


# speedup vs baseline: 2.6938x; 1.0615x over previous
import functools

import jax
import jax.numpy as jnp
from jax.experimental import pallas as pl

N = 50000
K = 256
Q_MIN = 0.1
S_B = 1.0
E_DEN_OFF = 1.0
BLK = 1000
NB = N // BLK
BIG = 2 ** 30


def _a1_body(tidx_ref, beta_ref, segmax_ref, pwsum_ref, nstat_ref):
    pid = pl.program_id(0)

    @pl.when(pid == 0)
    def _():
        segmax_ref[...] = jnp.full((1, K), -1.0, jnp.float32)
        pwsum_ref[...] = jnp.zeros((1, K), jnp.float32)
        nstat_ref[...] = jnp.zeros((1, 128), jnp.float32)

    tidx = tidx_ref[...]
    beta_raw = beta_ref[...]
    beta = jnp.clip(beta_raw, 1e-6, 1.0 - 1e-4)
    is_noise = tidx < 0
    cols = jax.lax.broadcasted_iota(jnp.int32, (1, K), 1)
    m = tidx == cols
    segmax_ref[...] = jnp.maximum(
        segmax_ref[...],
        jnp.max(jnp.where(m, beta, -1.0), axis=0, keepdims=True))
    pwsum_ref[...] += jnp.sum(jnp.where(m, beta * beta, 0.0),
                              axis=0, keepdims=True)
    nf = is_noise.astype(jnp.float32)
    lane = jax.lax.broadcasted_iota(jnp.int32, (1, 128), 1)
    upd = (jnp.where(lane == 0, jnp.sum(nf), 0.0)
           + jnp.where(lane == 1, jnp.sum(nf * beta), 0.0))
    nstat_ref[...] += upd


def _a2_body(tidx_ref, beta_ref, segmax_ref, aidx_ref):
    pid = pl.program_id(0)

    @pl.when(pid == 0)
    def _():
        aidx_ref[...] = jnp.full((1, K), BIG, jnp.int32)

    tidx = tidx_ref[...]
    beta_raw = beta_ref[...]
    beta = jnp.clip(beta_raw, 1e-6, 1.0 - 1e-4)
    cols = jax.lax.broadcasted_iota(jnp.int32, (1, K), 1)
    m = tidx == cols
    smax = jnp.sum(jnp.where(m, segmax_ref[...], 0.0), axis=1, keepdims=True)
    rows = jax.lax.broadcasted_iota(jnp.int32, (BLK, 1), 0)
    gidx = rows + pid * BLK
    key = jnp.where(beta >= smax, gidx, BIG)
    aidx_ref[...] = jnp.minimum(
        aidx_ref[...],
        jnp.min(jnp.where(m, key, BIG), axis=0, keepdims=True))


def _d_body(tidx_ref, beta_ref, cc_ref, en_ref, pos_ref, tim_ref,
            ten_ref, tpos_ref, ttim_ref,
            xa0_ref, xa1_ref, qa_ref, invw_ref,
            acc_ref):
    pid = pl.program_id(0)

    @pl.when(pid == 0)
    def _():
        acc_ref[...] = jnp.zeros((1, 128), jnp.float32)

    tidx = tidx_ref[...]
    beta_raw = beta_ref[...]
    beta = jnp.clip(beta_raw, 1e-6, 1.0 - 1e-4)
    q = 0.25 * jnp.log((1.0 + beta) / (1.0 - beta)) ** 2 + Q_MIN

    en = en_ref[...]
    ten = ten_ref[...]
    e_l = ((en - ten) / (jnp.abs(ten) + E_DEN_OFF)) ** 2
    dp = pos_ref[...] - tpos_ref[...]
    pos_d = jnp.sqrt(jnp.sum(dp * dp, axis=1, keepdims=True) + 1e-6)
    pos_l = jnp.where(pos_d < 10.0, pos_d * pos_d,
                      100.0 + 20.0 * (pos_d - 10.0))
    tim_l = (tim_ref[...] - ttim_ref[...]) ** 2
    w = beta * beta * (e_l + pos_l + tim_l)

    cols = jax.lax.broadcasted_iota(jnp.int32, (1, K), 1)
    m = tidx == cols
    cc = cc_ref[...]
    x0 = cc[:, 0:1]
    x1 = cc[:, 1:2]
    dx0 = x0 - xa0_ref[...]
    dx1 = x1 - xa1_ref[...]
    d2 = dx0 * dx0 + dx1 * dx1
    dist = jnp.sqrt(d2 + 1e-9)
    qq = q * qa_ref[...]
    pot = jnp.where(m,
                    qq * d2 + w * invw_ref[...],
                    qq * jnp.maximum(1.0 - dist, 0.0))
    lane = jax.lax.broadcasted_iota(jnp.int32, (1, 128), 1)
    acc_ref[...] += jnp.where(lane == 0, jnp.sum(pot), 0.0)


def _row_spec(width):
    return pl.BlockSpec((BLK, width), lambda i: (i, 0))


_FULL_K = pl.BlockSpec((1, K), lambda i: (0, 0))
_FULL_S = pl.BlockSpec((1, 128), lambda i: (0, 0))


@jax.jit
def _run(pred_beta, pred_ccoords, pred_energy, pred_pos, pred_time,
         t_idx, t_energy, t_pos, t_time):
    segmax, pwsum, nstat = pl.pallas_call(
        _a1_body,
        grid=(NB,),
        in_specs=[_row_spec(1), _row_spec(1)],
        out_specs=[_FULL_K, _FULL_K, _FULL_S],
        out_shape=[
            jax.ShapeDtypeStruct((1, K), jnp.float32),
            jax.ShapeDtypeStruct((1, K), jnp.float32),
            jax.ShapeDtypeStruct((1, 128), jnp.float32),
        ],
    )(t_idx, pred_beta)

    aidx = pl.pallas_call(
        _a2_body,
        grid=(NB,),
        in_specs=[_row_spec(1), _row_spec(1), _FULL_K],
        out_specs=_FULL_K,
        out_shape=jax.ShapeDtypeStruct((1, K), jnp.int32),
    )(t_idx, pred_beta, segmax)

    aidx_v = aidx[0]
    valid = (aidx_v < N).astype(jnp.float32)
    a = jnp.minimum(aidx_v, N - 1)
    x_a = pred_ccoords[a]
    beta_a = jnp.clip(pred_beta[a, 0], 1e-6, 1.0 - 1e-4)
    q_a = (jnp.arctanh(beta_a) ** 2 + Q_MIN) * valid
    n_obj = jnp.maximum(jnp.sum(valid), 1.0)
    invw = jnp.float32(N) / (n_obj * (pwsum[0] + 1e-9))

    acc = pl.pallas_call(
        _d_body,
        grid=(NB,),
        in_specs=[_row_spec(1), _row_spec(1), _row_spec(2), _row_spec(1),
                  _row_spec(2), _row_spec(1), _row_spec(1), _row_spec(2),
                  _row_spec(1), _FULL_K, _FULL_K, _FULL_K, _FULL_K],
        out_specs=_FULL_S,
        out_shape=jax.ShapeDtypeStruct((1, 128), jnp.float32),
    )(t_idx, pred_beta, pred_ccoords, pred_energy, pred_pos, pred_time,
      t_energy, t_pos, t_time,
      x_a[:, 0][None, :], x_a[:, 1][None, :], q_a[None, :], invw[None, :])

    l_beta = jnp.sum((1.0 - beta_a) * valid) / n_obj
    n_noise = jnp.maximum(nstat[0, 0], 1.0)
    l_noise = S_B * nstat[0, 1] / n_noise
    total = acc[0, 0] / jnp.float32(N) + l_beta + l_noise
    return jnp.reshape(total, [1])


def kernel(pred_beta, pred_ccoords, pred_energy, pred_pos, pred_time,
           rechit_energy, t_idx, t_energy, t_pos, t_time, row_splits):
    lossval = _run(pred_beta, pred_ccoords, pred_energy, pred_pos, pred_time,
                   t_idx, t_energy, t_pos, t_time)
    return (pred_beta, lossval)
